# baseline (speedup 1.0000x reference)
import functools

import jax
import jax.numpy as jnp
from jax import lax
from jax.experimental import pallas as pl
from jax.experimental.pallas import tpu as pltpu
from jax.experimental.pallas import tpu_sc as plsc

N, E, D_NODE, D_EDGE = 10000, 320000, 128, 16
NC, NS = 2, 16
C = 64
PT_ITERS = E // (NS * C)
PT_TAIL = (E // C) - PT_ITERS * NS
ROWS_PER_TILE = 624
ROW_REM = N - NS * ROWS_PER_TILE

mesh = plsc.VectorSubcoreMesh(core_axis_name="c", subcore_axis_name="s",
                              num_cores=NC, num_subcores=NS)


def _fused_body(x_hbm, src_hbm, dst_hbm, ea_hbm, accx_out, ed_out,
         src0, src1, dst0, dst1, rows0, rows1, ea0, ea1, msg0, msg1, acc_sh,
         si0, si1, sg, ss0, ss1, sl0, sl1, st0, st1):
    cid = lax.axis_index("c")
    sid = lax.axis_index("s")
    src_v = (src0, src1)
    dst_v = (dst0, dst1)
    rows_v = (rows0, rows1)
    ea_v = (ea0, ea1)
    msg_v = (msg0, msg1)
    sem_i = (si0, si1)
    sem_s = (ss0, ss1)
    sem_l = (sl0, sl1)
    sem_t = (st0, st1)

    count_col = jnp.where(lax.iota(jnp.int32, 16) == 0, 1.0, 0.0)

    def _zero_rows(i, _):
        for j in range(D_NODE // 16):
            rows0[i, pl.ds(j * 16, 16)] = jnp.zeros((16,), jnp.float32)
            msg0[i, pl.ds(j * 16, 16)] = jnp.zeros((16,), jnp.float32)
            msg1[i, pl.ds(j * 16, 16)] = jnp.zeros((16,), jnp.float32)
        return 0
    lax.fori_loop(0, C, _zero_rows, 0)

    r0 = sid * ROWS_PER_TILE
    for k in range(ROWS_PER_TILE // C):
        pltpu.sync_copy(rows0, acc_sh.at[pl.ds(r0 + k * C, C)])
    rem = ROWS_PER_TILE - (ROWS_PER_TILE // C) * C
    pltpu.sync_copy(rows0.at[pl.ds(0, rem)],
                    acc_sh.at[pl.ds(r0 + ROWS_PER_TILE - rem, rem)])

    @pl.when(sid == 0)
    def _():
        pltpu.sync_copy(rows0.at[pl.ds(0, ROW_REM)],
                        acc_sh.at[pl.ds(NS * ROWS_PER_TILE, ROW_REM)])

    plsc.subcore_barrier()

    def _setc(i, _):
        msg0[i, pl.ds(16, 16)] = count_col
        msg1[i, pl.ds(16, 16)] = count_col
        return 0
    lax.fori_loop(0, C, _setc, 0)

    def _off(j):
        return (sid + j * NS) * C

    iters = PT_ITERS + jnp.where(sid < PT_TAIL, 1, 0)

    @pl.when(cid == 0)
    def _gather_role():
        pltpu.sync_copy(src_hbm.at[pl.ds(_off(0), C)], src0)
        pltpu.sync_copy(dst_hbm.at[pl.ds(_off(0), C)], dst0)

        def _iter(j, _):
            p = j % 2
            for par in (0, 1):
                @pl.when(p == par)
                def _():
                    s_v, d_v, r_v = src_v[par], dst_v[par], rows_v[par]
                    s_vq, d_vq = src_v[1 - par], dst_v[1 - par]

                    @pl.when(j >= 1)
                    def _():
                        pltpu.make_async_copy(
                            src_hbm.at[pl.ds(_off(j), C)], s_v, sem_i[par]).wait()
                        pltpu.make_async_copy(
                            dst_hbm.at[pl.ds(_off(j), C)], d_v, sem_i[par]).wait()

                    pltpu.async_copy(x_hbm.at[s_v], r_v, sg)

                    @pl.when(j >= 1)
                    def _():
                        pltpu.make_async_copy(
                            rows_v[1 - par], acc_sh.at[d_vq], sem_s[1 - par]).wait()

                    @pl.when(j + 1 < iters)
                    def _():
                        pltpu.async_copy(
                            src_hbm.at[pl.ds(_off(j + 1), C)], s_vq, sem_i[1 - par])
                        pltpu.async_copy(
                            dst_hbm.at[pl.ds(_off(j + 1), C)], d_vq, sem_i[1 - par])

                    pltpu.make_async_copy(x_hbm.at[s_v], r_v, sg).wait()
                    pltpu.async_copy(r_v, acc_sh.at[d_v], sem_s[par], add=True)
            return 0
        lax.fori_loop(0, iters, _iter, 0)

        for par in (0, 1):
            @pl.when((iters - 1) % 2 == par)
            def _():
                pltpu.make_async_copy(rows_v[par], acc_sh.at[dst_v[par]],
                                      sem_s[par]).wait()

    @pl.when(cid == 1)
    def _edge_role():
        pltpu.sync_copy(dst_hbm.at[pl.ds(_off(0), C)], dst0)
        pltpu.sync_copy(ea_hbm.at[pl.ds(_off(0), C)], ea0)

        def _iter(j, _):
            p = j % 2
            for par in (0, 1):
                @pl.when(p == par)
                def _():
                    d_v, e_v, m_v = dst_v[par], ea_v[par], msg_v[par]
                    d_vq, e_vq = dst_v[1 - par], ea_v[1 - par]

                    @pl.when(j >= 1)
                    def _():
                        pltpu.make_async_copy(
                            dst_hbm.at[pl.ds(_off(j), C)], d_v, sem_l[par]).wait()
                        pltpu.make_async_copy(
                            ea_hbm.at[pl.ds(_off(j), C)], e_v, sem_l[par]).wait()

                    def _merge(i, _):
                        m_v[i, pl.ds(0, 16)] = e_v[i, :]
                        return 0
                    lax.fori_loop(0, C, _merge, 0)

                    @pl.when(j >= 1)
                    def _():
                        pltpu.make_async_copy(
                            msg_v[1 - par], acc_sh.at[d_vq], sem_t[1 - par]).wait()

                    @pl.when(j + 1 < iters)
                    def _():
                        pltpu.async_copy(
                            dst_hbm.at[pl.ds(_off(j + 1), C)], d_vq, sem_l[1 - par])
                        pltpu.async_copy(
                            ea_hbm.at[pl.ds(_off(j + 1), C)], e_vq, sem_l[1 - par])

                    pltpu.async_copy(m_v, acc_sh.at[d_v], sem_t[par], add=True)
            return 0
        lax.fori_loop(0, iters, _iter, 0)

        for par in (0, 1):
            @pl.when((iters - 1) % 2 == par)
            def _():
                pltpu.make_async_copy(msg_v[par], acc_sh.at[dst_v[par]],
                                      sem_t[par]).wait()

    plsc.subcore_barrier()

    @pl.when(cid == 0)
    def _():
        pltpu.sync_copy(acc_sh.at[pl.ds(r0, ROWS_PER_TILE)],
                        accx_out.at[pl.ds(r0, ROWS_PER_TILE)])

        @pl.when(sid == 0)
        def _():
            b = NS * ROWS_PER_TILE
            pltpu.sync_copy(acc_sh.at[pl.ds(b, ROW_REM)],
                            accx_out.at[pl.ds(b, ROW_REM)])

    @pl.when(cid == 1)
    def _():
        pltpu.sync_copy(acc_sh.at[pl.ds(r0, ROWS_PER_TILE)],
                        ed_out.at[pl.ds(r0, ROWS_PER_TILE)])

        @pl.when(sid == 0)
        def _():
            b = NS * ROWS_PER_TILE
            pltpu.sync_copy(acc_sh.at[pl.ds(b, ROW_REM)],
                            ed_out.at[pl.ds(b, ROW_REM)])


_sc_fused = functools.partial(
    pl.kernel,
    out_type=(jax.ShapeDtypeStruct((N, D_NODE), jnp.float32),
              jax.ShapeDtypeStruct((N, D_NODE), jnp.float32)),
    mesh=mesh,
    scratch_types=[
        pltpu.VMEM((C,), jnp.int32),
        pltpu.VMEM((C,), jnp.int32),
        pltpu.VMEM((C,), jnp.int32),
        pltpu.VMEM((C,), jnp.int32),
        pltpu.VMEM((C, D_NODE), jnp.float32),
        pltpu.VMEM((C, D_NODE), jnp.float32),
        pltpu.VMEM((C, D_EDGE), jnp.float32),
        pltpu.VMEM((C, D_EDGE), jnp.float32),
        pltpu.VMEM((C, D_NODE), jnp.float32),
        pltpu.VMEM((C, D_NODE), jnp.float32),
        pltpu.VMEM_SHARED((N, D_NODE), jnp.float32),
    ] + [pltpu.SemaphoreType.DMA] * 9,
)(_fused_body)


D_OUT = 128


def _tc_body(accx_ref, ed_ref, w_ref, b_ref, out_ref):
    sx = accx_ref[...]
    ed = ed_ref[...]
    se = ed[:, 0:D_EDGE]
    d = ed[:, D_EDGE:D_EDGE + 1]
    inv = 1.0 / jnp.maximum(d, 1.0)
    feat = jnp.dot(se, w_ref[pl.ds(0, D_EDGE), :],
                   preferred_element_type=jnp.float32)
    feat += jnp.dot(sx, w_ref[pl.ds(D_EDGE, D_NODE), :],
                    preferred_element_type=jnp.float32)
    out_ref[...] = jnp.maximum(feat * inv + b_ref[...], 0.0)


def kernel(x, edge_index, edge_attr, weight, bias):
    src = edge_index[0]
    dst = edge_index[1]
    accx, ed = _sc_fused(x, src, dst, edge_attr)
    out = pl.pallas_call(
        _tc_body,
        out_shape=jax.ShapeDtypeStruct((N, D_OUT), jnp.float32),
    )(accx, ed, weight, bias.reshape(1, D_OUT))
    return out


# speedup vs baseline: 1.2530x; 1.2530x over previous
import functools

import jax
import jax.numpy as jnp
from jax import lax
from jax.experimental import pallas as pl
from jax.experimental.pallas import tpu as pltpu
from jax.experimental.pallas import tpu_sc as plsc

N = 10000
E = 320000
D_NODE = 128
D_EDGE = 16
D_OUT = 128
W_ED = 128

NC = 2
NS = 16
NW = NC * NS
C = 80

NCHUNKS = E // C
FULL_ITERS = NCHUNKS // NW
TAIL = NCHUNKS - FULL_ITERS * NW

ROWS_PER_TILE = 624
ROW_REM = N - NS * ROWS_PER_TILE


def _sc_gather_body(x_hbm, src_hbm, dst_hbm, accx_out,
                    src0, src1, dst0, dst1, rows0, rows1, accx_sh,
                    si0, si1, sg, ss0, ss1):
    cid = lax.axis_index("c")
    sid = lax.axis_index("s")
    wid = sid * NC + cid
    src_v = (src0, src1)
    dst_v = (dst0, dst1)
    rows_v = (rows0, rows1)
    sem_i = (si0, si1)
    sem_s = (ss0, ss1)

    def _zero_rows(i, _):
        for j in range(D_NODE // 16):
            rows0[i, pl.ds(j * 16, 16)] = jnp.zeros((16,), jnp.float32)
        return 0
    lax.fori_loop(0, C, _zero_rows, 0)

    r0 = sid * ROWS_PER_TILE
    for k in range(ROWS_PER_TILE // C):
        pltpu.sync_copy(rows0, accx_sh.at[pl.ds(r0 + k * C, C)])
    rem = ROWS_PER_TILE - (ROWS_PER_TILE // C) * C
    pltpu.sync_copy(rows0.at[pl.ds(0, rem)],
                    accx_sh.at[pl.ds(r0 + ROWS_PER_TILE - rem, rem)])

    @pl.when(sid == 0)
    def _():
        pltpu.sync_copy(rows0.at[pl.ds(0, ROW_REM)],
                        accx_sh.at[pl.ds(NS * ROWS_PER_TILE, ROW_REM)])

    plsc.subcore_barrier()

    def _off(j):
        return (wid + j * NW) * C

    pltpu.sync_copy(src_hbm.at[pl.ds(_off(0), C)], src0)
    pltpu.sync_copy(dst_hbm.at[pl.ds(_off(0), C)], dst0)

    iters = FULL_ITERS + jnp.where(wid < TAIL, 1, 0)

    def _iter(j, _):
        p = j % 2
        for par in (0, 1):
            @pl.when(p == par)
            def _():
                s_v, d_v, r_v = src_v[par], dst_v[par], rows_v[par]
                s_vq, d_vq = src_v[1 - par], dst_v[1 - par]

                @pl.when(j >= 1)
                def _():
                    pltpu.make_async_copy(
                        src_hbm.at[pl.ds(_off(j), C)], s_v, sem_i[par]).wait()
                    pltpu.make_async_copy(
                        dst_hbm.at[pl.ds(_off(j), C)], d_v, sem_i[par]).wait()

                pltpu.async_copy(x_hbm.at[s_v], r_v, sg)

                @pl.when(j >= 1)
                def _():
                    pltpu.make_async_copy(
                        rows_v[1 - par], accx_sh.at[d_vq], sem_s[1 - par]).wait()

                @pl.when(j + 1 < iters)
                def _():
                    pltpu.async_copy(
                        src_hbm.at[pl.ds(_off(j + 1), C)], s_vq, sem_i[1 - par])
                    pltpu.async_copy(
                        dst_hbm.at[pl.ds(_off(j + 1), C)], d_vq, sem_i[1 - par])

                pltpu.make_async_copy(x_hbm.at[s_v], r_v, sg).wait()
                pltpu.async_copy(r_v, accx_sh.at[d_v], sem_s[par], add=True)
        return 0

    lax.fori_loop(0, iters, _iter, 0)

    for par in (0, 1):
        @pl.when((iters - 1) % 2 == par)
        def _():
            pltpu.make_async_copy(rows_v[par], accx_sh.at[dst_v[par]],
                                  sem_s[par]).wait()

    plsc.subcore_barrier()

    pltpu.sync_copy(accx_sh.at[pl.ds(r0, ROWS_PER_TILE)],
                    accx_out.at[pl.ds(cid * N + r0, ROWS_PER_TILE)])

    @pl.when(sid == 0)
    def _():
        b = NS * ROWS_PER_TILE
        pltpu.sync_copy(accx_sh.at[pl.ds(b, ROW_REM)],
                        accx_out.at[pl.ds(cid * N + b, ROW_REM)])


_sc_gather_x = functools.partial(
    pl.kernel,
    out_type=jax.ShapeDtypeStruct((NC * N, D_NODE), jnp.float32),
    mesh=plsc.VectorSubcoreMesh(core_axis_name="c", subcore_axis_name="s",
                                num_cores=NC, num_subcores=NS),
    scratch_types=[
        pltpu.VMEM((C,), jnp.int32),
        pltpu.VMEM((C,), jnp.int32),
        pltpu.VMEM((C,), jnp.int32),
        pltpu.VMEM((C,), jnp.int32),
        pltpu.VMEM((C, D_NODE), jnp.float32),
        pltpu.VMEM((C, D_NODE), jnp.float32),
        pltpu.VMEM_SHARED((N, D_NODE), jnp.float32),
        pltpu.SemaphoreType.DMA,
        pltpu.SemaphoreType.DMA,
        pltpu.SemaphoreType.DMA,
        pltpu.SemaphoreType.DMA,
        pltpu.SemaphoreType.DMA,
    ],
)(_sc_gather_body)


def _sc_edge_body(dst_hbm, ea_hbm, ed_out,
                  dst0, dst1, ea0, ea1, msg0, msg1, ed_sh,
                  sl0, sl1, ss0, ss1):
    cid = lax.axis_index("c")
    sid = lax.axis_index("s")
    wid = sid * NC + cid
    dst_v = (dst0, dst1)
    ea_v = (ea0, ea1)
    msg_v = (msg0, msg1)
    sem_l = (sl0, sl1)
    sem_s = (ss0, ss1)

    count_col = jnp.where(lax.iota(jnp.int32, 16) == 0, 1.0, 0.0)

    def _init_rows(i, _):
        for j in range(W_ED // 16):
            msg0[i, pl.ds(j * 16, 16)] = jnp.zeros((16,), jnp.float32)
            msg1[i, pl.ds(j * 16, 16)] = jnp.zeros((16,), jnp.float32)
        return 0
    lax.fori_loop(0, C, _init_rows, 0)

    r0 = sid * ROWS_PER_TILE
    for k in range(ROWS_PER_TILE // C):
        pltpu.sync_copy(msg0, ed_sh.at[pl.ds(r0 + k * C, C)])
    rem = ROWS_PER_TILE - (ROWS_PER_TILE // C) * C
    pltpu.sync_copy(msg0.at[pl.ds(0, rem)],
                    ed_sh.at[pl.ds(r0 + ROWS_PER_TILE - rem, rem)])

    @pl.when(sid == 0)
    def _():
        pltpu.sync_copy(msg0.at[pl.ds(0, ROW_REM)],
                        ed_sh.at[pl.ds(NS * ROWS_PER_TILE, ROW_REM)])

    plsc.subcore_barrier()

    def _set_count(i, _):
        msg0[i, pl.ds(16, 16)] = count_col
        msg1[i, pl.ds(16, 16)] = count_col
        return 0
    lax.fori_loop(0, C, _set_count, 0)

    def _off(j):
        return (wid + j * NW) * C

    pltpu.sync_copy(dst_hbm.at[pl.ds(_off(0), C)], dst0)
    pltpu.sync_copy(ea_hbm.at[pl.ds(_off(0), C)], ea0)

    iters = FULL_ITERS + jnp.where(wid < TAIL, 1, 0)

    def _iter(j, _):
        p = j % 2
        for par in (0, 1):
            @pl.when(p == par)
            def _():
                d_v, e_v, m_v = dst_v[par], ea_v[par], msg_v[par]
                d_vq, e_vq = dst_v[1 - par], ea_v[1 - par]

                @pl.when(j >= 1)
                def _():
                    pltpu.make_async_copy(
                        dst_hbm.at[pl.ds(_off(j), C)], d_v, sem_l[par]).wait()
                    pltpu.make_async_copy(
                        ea_hbm.at[pl.ds(_off(j), C)], e_v, sem_l[par]).wait()

                def _merge(i, _):
                    m_v[i, pl.ds(0, 16)] = e_v[i, :]
                    return 0
                lax.fori_loop(0, C, _merge, 0)

                @pl.when(j >= 1)
                def _():
                    pltpu.make_async_copy(
                        msg_v[1 - par], ed_sh.at[d_vq], sem_s[1 - par]).wait()

                @pl.when(j + 1 < iters)
                def _():
                    pltpu.async_copy(
                        dst_hbm.at[pl.ds(_off(j + 1), C)], d_vq, sem_l[1 - par])
                    pltpu.async_copy(
                        ea_hbm.at[pl.ds(_off(j + 1), C)], e_vq, sem_l[1 - par])

                pltpu.async_copy(m_v, ed_sh.at[d_v], sem_s[par], add=True)
        return 0

    lax.fori_loop(0, iters, _iter, 0)

    for par in (0, 1):
        @pl.when((iters - 1) % 2 == par)
        def _():
            pltpu.make_async_copy(msg_v[par], ed_sh.at[dst_v[par]],
                                  sem_s[par]).wait()

    plsc.subcore_barrier()

    pltpu.sync_copy(ed_sh.at[pl.ds(r0, ROWS_PER_TILE)],
                    ed_out.at[pl.ds(cid * N + r0, ROWS_PER_TILE)])

    @pl.when(sid == 0)
    def _():
        b = NS * ROWS_PER_TILE
        pltpu.sync_copy(ed_sh.at[pl.ds(b, ROW_REM)],
                        ed_out.at[pl.ds(cid * N + b, ROW_REM)])


_sc_edge = functools.partial(
    pl.kernel,
    out_type=jax.ShapeDtypeStruct((NC * N, W_ED), jnp.float32),
    mesh=plsc.VectorSubcoreMesh(core_axis_name="c", subcore_axis_name="s",
                                num_cores=NC, num_subcores=NS),
    scratch_types=[
        pltpu.VMEM((C,), jnp.int32),
        pltpu.VMEM((C,), jnp.int32),
        pltpu.VMEM((C, D_EDGE), jnp.float32),
        pltpu.VMEM((C, D_EDGE), jnp.float32),
        pltpu.VMEM((C, W_ED), jnp.float32),
        pltpu.VMEM((C, W_ED), jnp.float32),
        pltpu.VMEM_SHARED((N, W_ED), jnp.float32),
        pltpu.SemaphoreType.DMA,
        pltpu.SemaphoreType.DMA,
        pltpu.SemaphoreType.DMA,
        pltpu.SemaphoreType.DMA,
    ],
)(_sc_edge_body)


def _tc_body(accx_ref, ed_ref, w_ref, b_ref, out_ref):
    sx = accx_ref[pl.ds(0, N), :] + accx_ref[pl.ds(N, N), :]
    ed = ed_ref[pl.ds(0, N), :] + ed_ref[pl.ds(N, N), :]
    se = ed[:, 0:D_EDGE]
    d = ed[:, D_EDGE:D_EDGE + 1]
    inv = 1.0 / jnp.maximum(d, 1.0)
    feat = jnp.dot(se, w_ref[pl.ds(0, D_EDGE), :],
                   preferred_element_type=jnp.float32)
    feat += jnp.dot(sx, w_ref[pl.ds(D_EDGE, D_NODE), :],
                    preferred_element_type=jnp.float32)
    out_ref[...] = jnp.maximum(feat * inv + b_ref[...], 0.0)


def kernel(x, edge_index, edge_attr, weight, bias):
    src = edge_index[0]
    dst = edge_index[1]
    accx = _sc_gather_x(x, src, dst)
    ed = _sc_edge(dst, edge_attr)
    out = pl.pallas_call(
        _tc_body,
        out_shape=jax.ShapeDtypeStruct((N, D_OUT), jnp.float32),
    )(accx, ed, weight, bias.reshape(1, D_OUT))
    return out
